# speedup vs baseline: 1.0370x; 1.0370x over previous
import functools

import jax
import jax.numpy as jnp
from jax import lax
from jax.experimental import pallas as pl
from jax.experimental.pallas import tpu as pltpu
from jax.experimental.pallas import tpu_sc as plsc

_INFO = plsc.get_sparse_core_info()
_NC = _INFO.num_cores
_NS = _INFO.num_subcores
_NW = _NC * _NS
_L = _INFO.num_lanes


def _build(total_rows, seq, hidden):
    rpw = total_rows // _NW
    chunk = 16
    nch = rpw // chunk
    nv = hidden // _L
    mesh = plsc.VectorSubcoreMesh(core_axis_name="c", subcore_axis_name="s")

    def body(tok_hbm, ids_hbm, pos_hbm, out_hbm,
             idx_v, tok_buf, pos_buf, out_buf, tok_sem, pos_sem, out_sem):
        cid = lax.axis_index("c")
        sid = lax.axis_index("s")
        wid = sid * _NC + cid
        base = wid * rpw
        pos_base = lax.rem(base, seq)

        pltpu.sync_copy(ids_hbm.at[pl.ds(base, rpw)], idx_v)

        def issue(c):
            slot = c & 1
            t = pltpu.async_copy(
                tok_hbm.at[idx_v.at[pl.ds(c * chunk, chunk)]],
                tok_buf.at[slot], tok_sem.at[slot])
            p = pltpu.async_copy(
                pos_hbm.at[pl.ds(pos_base + c * chunk, chunk)],
                pos_buf.at[slot], pos_sem.at[slot])
            return t, p

        pend = [issue(0), issue(1)]
        out_pend = [None, None]
        for c in range(nch):
            slot = c & 1
            t, p = pend[slot]
            t.wait()
            p.wait()
            if out_pend[slot] is not None:
                out_pend[slot].wait()

            def row(r, _, slot=slot):
                for k in range(nv):
                    sl = pl.ds(k * _L, _L)
                    out_buf[slot, r, sl] = (
                        tok_buf[slot, r, sl] + pos_buf[slot, r, sl])
                return 0

            lax.fori_loop(0, chunk, row, 0)

            out_pend[slot] = pltpu.async_copy(
                out_buf.at[slot],
                out_hbm.at[pl.ds(base + c * chunk, chunk)],
                out_sem.at[slot])
            if c + 2 < nch:
                pend[slot] = issue(c + 2)
        out_pend[0].wait()
        out_pend[1].wait()

    return pl.kernel(
        body,
        out_type=jax.ShapeDtypeStruct((total_rows, hidden), jnp.float32),
        mesh=mesh,
        scratch_types=[
            pltpu.VMEM((rpw,), jnp.int32),
            pltpu.VMEM((2, chunk, hidden), jnp.float32),
            pltpu.VMEM((2, chunk, hidden), jnp.float32),
            pltpu.VMEM((2, chunk, hidden), jnp.float32),
            pltpu.SemaphoreType.DMA((2,)),
            pltpu.SemaphoreType.DMA((2,)),
            pltpu.SemaphoreType.DMA((2,)),
        ],
    )


@jax.jit
def kernel(input_ids, token_table, pos_table):
    b, seq = input_ids.shape
    hidden = token_table.shape[1]
    ids = input_ids.reshape(-1).astype(jnp.int32)
    out = _build(b * seq, seq, hidden)(token_table, ids, pos_table)
    return out.reshape(b, seq, hidden)


# speedup vs baseline: 1.0958x; 1.0567x over previous
import functools

import jax
import jax.numpy as jnp
from jax import lax
from jax.experimental import pallas as pl
from jax.experimental.pallas import tpu as pltpu
from jax.experimental.pallas import tpu_sc as plsc

_INFO = plsc.get_sparse_core_info()
_NC = _INFO.num_cores
_NS = _INFO.num_subcores
_NW = _NC * _NS
_L = _INFO.num_lanes


def _build(batch, seq, hidden):
    spw = seq // _NW
    chunk = 16
    cpb = spw // chunk
    nch = batch * cpb
    ring = 3
    nv = hidden // _L
    mesh = plsc.VectorSubcoreMesh(core_axis_name="c", subcore_axis_name="s")

    def body(tok_hbm, ids_hbm, pos_hbm, out_hbm,
             idx_v, pos_res, tok_buf, idx_sem, pos_sem, tok_sem, out_sem):
        cid = lax.axis_index("c")
        sid = lax.axis_index("s")
        wid = sid * _NC + cid
        s_base = wid * spw

        idx_d = [
            pltpu.async_copy(
                ids_hbm.at[pl.ds(b * seq + s_base, spw)],
                idx_v.at[pl.ds(b * spw, spw)], idx_sem.at[b])
            for b in range(batch)
        ]
        pos_d = pltpu.async_copy(
            pos_hbm.at[pl.ds(s_base, spw)], pos_res, pos_sem)
        for d in idx_d:
            d.wait()

        def issue_gather(c):
            s = c % ring
            return pltpu.async_copy(
                tok_hbm.at[idx_v.at[pl.ds(c * chunk, chunk)]],
                tok_buf.at[s], tok_sem.at[s])

        def issue_out(c):
            s = c % ring
            b, j = divmod(c, cpb)
            return pltpu.async_copy(
                tok_buf.at[s],
                out_hbm.at[pl.ds(b * seq + s_base + j * chunk, chunk)],
                out_sem.at[s])

        gad_d = [None] * nch
        out_d = [None] * nch
        waited_pos = False
        for i in range(nch + 1):
            if i < nch:
                if i >= ring:
                    out_d[i - ring].wait()
                gad_d[i] = issue_gather(i)
            c = i - 1
            if c >= 0:
                s = c % ring
                j = c % cpb
                gad_d[c].wait()
                if not waited_pos:
                    pos_d.wait()
                    waited_pos = True

                def row(r, _, s=s, j=j):
                    for k in range(nv):
                        sl = pl.ds(k * _L, _L)
                        tok_buf[s, r, sl] = (
                            tok_buf[s, r, sl] + pos_res[j * chunk + r, sl])
                    return 0

                lax.fori_loop(0, chunk, row, 0)
                out_d[c] = issue_out(c)
        for c in range(nch - ring, nch):
            out_d[c].wait()

    return pl.kernel(
        body,
        out_type=jax.ShapeDtypeStruct((batch * seq, hidden), jnp.float32),
        mesh=mesh,
        scratch_types=[
            pltpu.VMEM((batch * spw,), jnp.int32),
            pltpu.VMEM((spw, hidden), jnp.float32),
            pltpu.VMEM((ring, chunk, hidden), jnp.float32),
            pltpu.SemaphoreType.DMA((batch,)),
            pltpu.SemaphoreType.DMA,
            pltpu.SemaphoreType.DMA((ring,)),
            pltpu.SemaphoreType.DMA((ring,)),
        ],
    )


@jax.jit
def kernel(input_ids, token_table, pos_table):
    b, seq = input_ids.shape
    hidden = token_table.shape[1]
    ids = input_ids.reshape(-1).astype(jnp.int32)
    out = _build(b, seq, hidden)(token_table, ids, pos_table)
    return out.reshape(b, seq, hidden)


# speedup vs baseline: 1.4984x; 1.3674x over previous
import functools

import jax
import jax.numpy as jnp
from jax import lax
from jax.experimental import pallas as pl
from jax.experimental.pallas import tpu as pltpu
from jax.experimental.pallas import tpu_sc as plsc

_INFO = plsc.get_sparse_core_info()
_NC = _INFO.num_cores
_NS = _INFO.num_subcores
_NW = _NC * _NS
_L = _INFO.num_lanes


def _build(batch, seq, hidden):
    spw = seq // _NW
    q = 8
    ng = spw // q
    grows = batch * q
    rpw = batch * spw
    ring = 3
    nv = hidden // _L
    mesh = plsc.VectorSubcoreMesh(core_axis_name="c", subcore_axis_name="s")

    def body(tok_hbm, ids_hbm, pos_hbm, out_hbm,
             idx_v, pos_buf, tok_buf, pos_sem, gad_sem, out_sem):
        cid = lax.axis_index("c")
        sid = lax.axis_index("s")
        wid = sid * _NC + cid
        s_base = wid * spw

        pltpu.sync_copy(ids_hbm.at[pl.ds(wid * rpw, rpw)], idx_v)

        def _gather_desc(j):
            ts = lax.rem(j, ring)
            return pltpu.make_async_copy(
                tok_hbm.at[idx_v.at[pl.ds(j * grows, grows)]],
                tok_buf.at[ts], gad_sem.at[ts])

        def _pos_desc(j):
            ps = lax.rem(j, ring)
            return pltpu.make_async_copy(
                pos_hbm.at[pl.ds(s_base + j * q, q)],
                pos_buf.at[ps], pos_sem.at[ps])

        def _scatter_descs(j):
            ts = lax.rem(j, ring)
            return [
                pltpu.make_async_copy(
                    tok_buf.at[ts, pl.ds(b * q, q)],
                    out_hbm.at[pl.ds(b * seq + s_base + j * q, q)],
                    out_sem.at[ts * batch + b])
                for b in range(batch)
            ]

        def gather(j):
            _gather_desc(j).start()

        def pos_load(j):
            _pos_desc(j).start()

        def scatter(j):
            for d in _scatter_descs(j):
                d.start()

        gather(0)
        pos_load(0)
        pos_load(1)

        def group(j, _):
            ts = lax.rem(j, ring)

            @pl.when(j + 1 < ng)
            def _():
                @pl.when(j >= 2)
                def _():
                    for d in _scatter_descs(j - 2):
                        d.wait()
                gather(j + 1)

                @pl.when(j + 2 < ng)
                def _():
                    pos_load(j + 2)

            _gather_desc(j).wait()
            _pos_desc(j).wait()

            def row(r, _):
                for k in range(nv):
                    sl = pl.ds(k * _L, _L)
                    p = pos_buf[ts, r, sl]
                    for b in range(batch):
                        tok_buf[ts, b * q + r, sl] = (
                            tok_buf[ts, b * q + r, sl] + p)
                return 0

            lax.fori_loop(0, q, row, 0)
            scatter(j)
            return 0

        lax.fori_loop(0, ng, group, 0)
        for j in (ng - 2, ng - 1):
            for d in _scatter_descs(j):
                d.wait()

    return pl.kernel(
        body,
        out_type=jax.ShapeDtypeStruct((batch * seq, hidden), jnp.float32),
        mesh=mesh,
        scratch_types=[
            pltpu.VMEM((rpw,), jnp.int32),
            pltpu.VMEM((ring, q, hidden), jnp.float32),
            pltpu.VMEM((ring, grows, hidden), jnp.float32),
            pltpu.SemaphoreType.DMA((ring,)),
            pltpu.SemaphoreType.DMA((ring,)),
            pltpu.SemaphoreType.DMA((ring * batch,)),
        ],
    )


@jax.jit
def kernel(input_ids, token_table, pos_table):
    b, seq = input_ids.shape
    hidden = token_table.shape[1]
    spw = seq // _NW
    q = 8
    ng = spw // q
    ids = (input_ids.astype(jnp.int32)
           .reshape(b, _NW, ng, q)
           .transpose(1, 2, 0, 3)
           .reshape(-1))
    out = _build(b, seq, hidden)(token_table, ids, pos_table)
    return out.reshape(b, seq, hidden)
